# baseline (speedup 1.0000x reference)
import jax
import jax.numpy as jnp
from jax import lax
from jax.experimental import pallas as pl
from jax.experimental.pallas import tpu as pltpu
from jax.experimental.pallas import tpu_sc as plsc

N_NODES = 10000
N_EDGES = 320000
D = 128

NC = 2
NS = 16
NW = NC * NS
EH0 = 128000
EH1 = N_EDGES - EH0
C = 40
NP = 10112
N_PER_TILE = NP // NS


def _preproj_body(x_ref, w1_ref, b1_ref, out_ref):
    xa = jnp.dot(x_ref[...], w1_ref[:D, :], preferred_element_type=jnp.float32)
    xb = jnp.dot(x_ref[...], w1_ref[D:, :], preferred_element_type=jnp.float32)
    out_ref[0] = xa
    out_ref[1] = xb + b1_ref[...]


def _preproj(x, W1, b1):
    BN = 2000
    grid = (N_NODES // BN,)
    return pl.pallas_call(
        _preproj_body,
        grid=grid,
        in_specs=[
            pl.BlockSpec((BN, D), lambda i: (i, 0)),
            pl.BlockSpec((2 * D, D), lambda i: (0, 0)),
            pl.BlockSpec((1, D), lambda i: (0, 0)),
        ],
        out_specs=pl.BlockSpec((2, BN, D), lambda i: (0, i, 0)),
        out_shape=jax.ShapeDtypeStruct((2, N_NODES, D), jnp.float32),
    )(x, W1, b1.reshape(1, D))


NBUF = 4
NBUF2 = 5


DW = D // 2
PC = 125
TROWS = 2 * N_NODES // NS


def _gather_body(do_pack, epw, tab_hbm, ridx_hbm, cidx_hbm, out_hbm,
                 ptab_hbm, ia_v, ib_v, tf_v, tp_v, ba, bb, bo, ssa):
    nch = epw // C
    cid = lax.axis_index("c")
    sid = lax.axis_index("s")
    wid = sid * NC + cid
    e0 = wid * epw

    if do_pack:
        tr0 = sid * TROWS

        def pack_chunk(k, c2):
            r0 = tr0 + k * PC
            pltpu.sync_copy(tab_hbm.at[pl.ds(r0, PC)], tf_v)

            def prow(i, c3):
                for j in range(D // 32):
                    a = tf_v[i, pl.ds(j * 32, 16)]
                    b = tf_v[i, pl.ds(j * 32 + 16, 16)]
                    w = plsc.bitcast(
                        plsc.pack(a, b, format=plsc.PackFormat.INTERLEAVED),
                        jnp.int32)
                    tp_v[i, pl.ds(j * 16, 16)] = w
                return c3

            lax.fori_loop(0, PC, prow, 0, unroll=2)
            pltpu.sync_copy(tp_v, ptab_hbm.at[cid, pl.ds(r0, PC)])
            return c2

        lax.fori_loop(0, TROWS // PC, pack_chunk, 0)
        plsc.subcore_barrier()

    ptab = ptab_hbm.at[cid]

    pltpu.sync_copy(ridx_hbm.at[pl.ds(e0, epw)], ia_v)
    pltpu.sync_copy(cidx_hbm.at[pl.ds(e0, epw)], ib_v)

    def issue_gather(ci, b):
        pltpu.async_copy(ptab.at[ia_v.at[pl.ds(ci * C, C)]], ba[b], ssa[b])
        pltpu.async_copy(ptab.at[ib_v.at[pl.ds(ci * C, C)]], bb[b], ssa[b])

    def wait_gather(b):
        pltpu.make_async_copy(ptab.at[pl.ds(0, C)], ba[b], ssa[b]).wait()
        pltpu.make_async_copy(ptab.at[pl.ds(0, C)], bb[b], ssa[b]).wait()

    def add_unpack(b):
        A, B, O = ba[b], bb[b], bo[b]

        def row_add(i, c2):
            for j in range(DW // 16):
                sl = pl.ds(j * 16, 16)
                s = (plsc.bitcast(A[i, sl], jnp.bfloat16)
                     + plsc.bitcast(B[i, sl], jnp.bfloat16))
                lo, hi = plsc.unpack(s, format=plsc.PackFormat.INTERLEAVED)
                O[i, pl.ds(j * 32, 16)] = lo
                O[i, pl.ds(j * 32 + 16, 16)] = hi
            return c2

        lax.fori_loop(0, C, row_add, 0, unroll=4)

    def issue_store(ci, b):
        base = e0 + ci * C
        pltpu.async_copy(bo[b], out_hbm.at[pl.ds(base, C)], ssa[NBUF2 + b])

    def drain_store(b):
        pltpu.make_async_copy(bo[b], out_hbm.at[pl.ds(0, C)],
                              ssa[NBUF2 + b]).wait()

    for b in range(NBUF2):
        issue_store(b, b)

    def body(k, carry):
        g = k * NBUF2
        for b in range(NBUF2):
            issue_gather(g + b, b)
        for b in range(NBUF2):
            wait_gather(b)
            drain_store(b)
            add_unpack(b)
            issue_store(g + b, b)
        return carry

    lax.fori_loop(0, nch // NBUF2, body, 0)
    for b in range(NBUF2):
        drain_store(b)


def _gather_scratch(epw):
    return (
        [pltpu.VMEM((epw,), jnp.int32)] * 2
        + [pltpu.VMEM((PC, D), jnp.float32),
           pltpu.VMEM((PC, DW), jnp.int32)]
        + [pltpu.VMEM((C, DW), jnp.int32)] * (2 * NBUF2)
        + [pltpu.VMEM((C, D), jnp.float32)] * NBUF2
        + [pltpu.SemaphoreType.DMA] * (2 * NBUF2)
    )


_SC_PARAMS = pltpu.CompilerParams(use_tc_tiling_on_sc=False,
                                  needs_layout_passes=False)


def _gather_pairs_pack(xab, ridx, cidxp):
    ne = ridx.shape[0]
    epw = ne // NW
    mesh = plsc.VectorSubcoreMesh(core_axis_name="c", subcore_axis_name="s")
    f = pl.kernel(
        lambda tab, ri, ci, out, ptab, iav, ibv, tfv, tpv, *rest: _gather_body(
            True, epw, tab, ri, ci, out, ptab, iav, ibv, tfv, tpv,
            rest[0:NBUF2], rest[NBUF2:2 * NBUF2],
            rest[2 * NBUF2:3 * NBUF2], rest[3 * NBUF2:]),
        out_type=(jax.ShapeDtypeStruct((ne, D), jnp.float32),
                  jax.ShapeDtypeStruct((NC, 2 * N_NODES, DW), jnp.int32)),
        mesh=mesh,
        compiler_params=_SC_PARAMS,
        scratch_types=_gather_scratch(epw),
    )
    return f(xab, ridx, cidxp)


def _gather_pairs_reuse(ptab, ridx, cidxp):
    ne = ridx.shape[0]
    epw = ne // NW
    mesh = plsc.VectorSubcoreMesh(core_axis_name="c", subcore_axis_name="s")
    f = pl.kernel(
        lambda pt, ri, ci, out, iav, ibv, tfv, tpv, *rest: _gather_body(
            False, epw, None, ri, ci, out, pt, iav, ibv, tfv, tpv,
            rest[0:NBUF2], rest[NBUF2:2 * NBUF2],
            rest[2 * NBUF2:3 * NBUF2], rest[3 * NBUF2:]),
        out_type=jax.ShapeDtypeStruct((ne, D), jnp.float32),
        mesh=mesh,
        compiler_params=_SC_PARAMS,
        scratch_types=_gather_scratch(epw),
    )
    return f(ptab, ridx, cidxp)


def _softplus(h):
    return jnp.maximum(h, 0.0) + jnp.log(1.0 + jnp.exp(-jnp.abs(h)))


def _mlp_body(h_ref, w2_ref, b2_ref, w3_ref, b3_ref, out_ref):
    h = _softplus(h_ref[...]).astype(jnp.bfloat16)
    h = _softplus(jnp.dot(h, w2_ref[...], preferred_element_type=jnp.float32)
                  + b2_ref[...]).astype(jnp.bfloat16)
    out_ref[...] = (jnp.dot(h, w3_ref[...], preferred_element_type=jnp.float32)
                    + b3_ref[...])


def _mlp_tail(hpre, W2, b2, W3, b3):
    BE = 3200
    ne = hpre.shape[0]
    grid = (ne // BE,)
    return pl.pallas_call(
        _mlp_body,
        grid=grid,
        in_specs=[
            pl.BlockSpec((BE, D), lambda i: (i, 0)),
            pl.BlockSpec((D, D), lambda i: (0, 0)),
            pl.BlockSpec((1, D), lambda i: (0, 0)),
            pl.BlockSpec((D, D), lambda i: (0, 0)),
            pl.BlockSpec((1, D), lambda i: (0, 0)),
        ],
        out_specs=pl.BlockSpec((BE, D), lambda i: (i, 0)),
        out_shape=jax.ShapeDtypeStruct((ne, D), jnp.float32),
    )(hpre, W2.astype(jnp.bfloat16), b2.reshape(1, D),
      W3.astype(jnp.bfloat16), b3.reshape(1, D))


ZR = 64


def _scatter_body(epw, h3_hbm, ridx3_hbm, out_hbm, ix2_v, bf, zbuf_v,
                  accum_sh, slh, ss):
    nch = epw // C
    cid = lax.axis_index("c")
    sid = lax.axis_index("s")
    wid = sid * NC + cid

    zeros16 = jnp.zeros((16,), jnp.float32)
    for i in range(ZR):
        for j in range(D // 16):
            zbuf_v[i, pl.ds(j * 16, 16)] = zeros16
    r0 = sid * N_PER_TILE
    for k in range(N_PER_TILE // ZR):
        pltpu.sync_copy(zbuf_v, accum_sh.at[pl.ds(r0 + k * ZR, ZR)])
    rem = N_PER_TILE % ZR
    pltpu.sync_copy(zbuf_v.at[pl.ds(0, rem)],
                    accum_sh.at[pl.ds(r0 + N_PER_TILE - rem, rem)])
    pltpu.sync_copy(ridx3_hbm.at[wid], ix2_v)
    plsc.subcore_barrier()

    e0 = wid * epw

    def issue_load(ci, b):
        base = e0 + ci * C
        pltpu.async_copy(h3_hbm.at[pl.ds(base, C)], bf[b], slh[b])

    def fire(ci, b):
        pltpu.make_async_copy(h3_hbm.at[pl.ds(0, C)], bf[b], slh[b]).wait()
        pltpu.async_copy(bf[b], accum_sh.at[ix2_v.at[ci]], ss[b], add=True)

    def drain(b):
        pltpu.make_async_copy(h3_hbm.at[pl.ds(0, C)], bf[b], ss[b]).wait()

    for b in range(NBUF):
        issue_load(b, b)

    def body(k, carry):
        g = k * NBUF
        for b in range(NBUF):
            fire(g + b, b)
        for b in range(NBUF):
            drain(b)

            @pl.when(g + NBUF + b < nch)
            def _():
                issue_load(g + NBUF + b, b)

        return carry

    nsup = nch // NBUF
    lax.fori_loop(0, nsup, body, 0)
    for t in range(nch % NBUF):
        fire(nsup * NBUF + t, t)
    for t in range(nch % NBUF):
        drain(t)
    plsc.subcore_barrier()

    pltpu.sync_copy(accum_sh.at[pl.ds(r0, N_PER_TILE)],
                    out_hbm.at[cid, pl.ds(r0, N_PER_TILE)])


def _segment_sum(h3, ridx3):
    epw = ridx3.shape[1] * C
    mesh = plsc.VectorSubcoreMesh(core_axis_name="c", subcore_axis_name="s")
    f = pl.kernel(
        lambda h3r, rir, out, *rest: _scatter_body(
            epw, h3r, rir, out,
            rest[0], rest[1:1 + NBUF],
            rest[1 + NBUF], rest[2 + NBUF],
            rest[3 + NBUF:3 + 2 * NBUF],
            rest[3 + 2 * NBUF:3 + 3 * NBUF]),
        out_type=jax.ShapeDtypeStruct((NC, NP, D), jnp.float32),
        mesh=mesh,
        scratch_types=(
            [pltpu.VMEM((epw // C, C), jnp.int32)]
            + [pltpu.VMEM((C, D), jnp.float32)] * NBUF
            + [pltpu.VMEM((ZR, D), jnp.float32),
               pltpu.VMEM_SHARED((NP, D), jnp.float32)]
            + [pltpu.SemaphoreType.DMA] * (2 * NBUF)
        ),
    )
    return f(h3, ridx3)


def _sum2_body(pa_ref, pb_ref, o_ref):
    o_ref[...] = (pa_ref[0] + pa_ref[1]) + (pb_ref[0] + pb_ref[1])


def _sum_partials(pa, pb):
    BN = 632
    grid = (NP // BN,)
    return pl.pallas_call(
        _sum2_body,
        grid=grid,
        in_specs=[pl.BlockSpec((2, BN, D), lambda i: (0, i, 0)),
                  pl.BlockSpec((2, BN, D), lambda i: (0, i, 0))],
        out_specs=pl.BlockSpec((BN, D), lambda i: (i, 0)),
        out_shape=jax.ShapeDtypeStruct((NP, D), jnp.float32),
    )(pa, pb)


def kernel(x, edge_idx, W1, b1, W2, b2, W3, b3):
    ridx = edge_idx[0].astype(jnp.int32)
    cidxp = edge_idx[1].astype(jnp.int32) + N_NODES

    xab = _preproj(x, W1, b1).reshape(2 * N_NODES, D)
    r0 = lax.dynamic_slice_in_dim(ridx, 0, EH0)
    c0 = lax.dynamic_slice_in_dim(cidxp, 0, EH0)
    r1 = lax.dynamic_slice_in_dim(ridx, EH0, EH1)
    c1 = lax.dynamic_slice_in_dim(cidxp, EH0, EH1)

    hpre0, ptab = _gather_pairs_pack(xab, r0, c0)
    h3_0 = _mlp_tail(hpre0, W2, b2, W3, b3)
    hpre1 = _gather_pairs_reuse(ptab, r1, c1)
    p0 = _segment_sum(h3_0, r0.reshape(NW, EH0 // NW // C, C))
    h3_1 = _mlp_tail(hpre1, W2, b2, W3, b3)
    p1 = _segment_sum(h3_1, r1.reshape(NW, EH1 // NW // C, C))
    return _sum_partials(p0, p1)[:N_NODES]
